# Initial kernel scaffold; baseline (speedup 1.0000x reference)
import jax
import jax.numpy as jnp
from jax.experimental import pallas as pl


def kernel(x, edge_index, W1l, b1, W1r, W2l, b2, W2r, W3l, b3, W3r):
    raise NotImplementedError("write your pallas kernel here")


# speedup vs baseline: 5.3272x; 5.3272x over previous
import functools

import jax
import jax.numpy as jnp
from jax import lax
from jax.experimental import pallas as pl
from jax.experimental.pallas import tpu as pltpu
from jax.experimental.pallas import tpu_sc as plsc

F32 = jnp.float32

N = 10000
E = 320000
F = 128
H = 128
C = 40
CP = 48

NC = 2
NS = 16
NW = NC * NS
K = 128
EW = -(-E // NW)
CH = -(-EW // K)
EWP = CH * K
RPT = ((N // NS) // 8 + 1) * 8
NPAD = RPT * NS

BN = 1000
NB = N // BN


def _make_segsum(width):
    mesh = plsc.VectorSubcoreMesh(core_axis_name="c", subcore_axis_name="s")
    out_type = jax.ShapeDtypeStruct((NC, NPAD, width), F32)
    scratch = [
        pltpu.VMEM((CH, K), jnp.int32),
        pltpu.VMEM((CH, K), jnp.int32),
        pltpu.VMEM((K, width), F32),
        pltpu.VMEM_SHARED((NPAD, width), F32),
        pltpu.SemaphoreType.DMA,
    ]

    def body(y_hbm, src_hbm, dst_hbm, zeros_hbm, out_hbm,
             src_v, dst_v, rows_v, agg_sh, gsem):
        cid = lax.axis_index("c")
        sid = lax.axis_index("s")
        wid = cid * NS + sid

        pltpu.sync_copy(zeros_hbm, agg_sh.at[pl.ds(sid * RPT, RPT)])
        pltpu.sync_copy(src_hbm.at[wid], src_v)
        pltpu.sync_copy(dst_hbm.at[wid], dst_v)
        plsc.subcore_barrier()

        def step(c, carry):
            pltpu.async_copy(y_hbm.at[src_v.at[c]], rows_v, gsem).wait()
            pltpu.sync_copy(rows_v, agg_sh.at[dst_v.at[c]], add=True)
            return carry

        lax.fori_loop(0, CH, step, 0)
        plsc.subcore_barrier()

        sl = pl.ds(sid * RPT, RPT)
        pltpu.sync_copy(agg_sh.at[sl], out_hbm.at[cid].at[sl])

    params = None
    if width % 128 != 0:
        params = pltpu.CompilerParams(use_tc_tiling_on_sc=False)
    return pl.kernel(body, out_type=out_type, mesh=mesh,
                     scratch_types=scratch, compiler_params=params)


def _make_deg():
    mesh = plsc.VectorSubcoreMesh(core_axis_name="c", subcore_axis_name="s")

    def body(dst_hbm, zeros16_hbm, ones_hbm, deg_hbm,
             dst_v, ones_v, deg_sh):
        cid = lax.axis_index("c")
        sid = lax.axis_index("s")
        wid = cid * NS + sid
        pltpu.sync_copy(zeros16_hbm, deg_sh.at[pl.ds(sid * RPT, RPT)])
        pltpu.sync_copy(ones_hbm, ones_v)
        pltpu.sync_copy(dst_hbm.at[wid], dst_v)
        plsc.subcore_barrier()

        def step(c, carry):
            pltpu.sync_copy(ones_v, deg_sh.at[dst_v.at[c]], add=True)
            return carry

        lax.fori_loop(0, CH, step, 0)
        plsc.subcore_barrier()
        sl = pl.ds(sid * RPT, RPT)
        pltpu.sync_copy(deg_sh.at[sl], deg_hbm.at[cid].at[sl])

    return pl.kernel(
        body,
        out_type=jax.ShapeDtypeStruct((NC, NPAD, 16), F32),
        mesh=mesh,
        scratch_types=[
            pltpu.VMEM((CH, K), jnp.int32),
            pltpu.VMEM((K, 16), F32),
            pltpu.VMEM_SHARED((NPAD, 16), F32),
        ],
        compiler_params=pltpu.CompilerParams(use_tc_tiling_on_sc=False),
    )


_segsum_h = _make_segsum(H)
_segsum_c = _make_segsum(CP)
_deg_kernel = _make_deg()


def _entry_body(x_ref, w_ref, y_ref, z_ref):
    acc = jnp.dot(x_ref[...], w_ref[...], preferred_element_type=F32)
    y_ref[...] = acc[:, :H]
    z_ref[...] = acc[:, H:]


def _entry(x, wcat):
    return pl.pallas_call(
        _entry_body,
        grid=(NB,),
        in_specs=[
            pl.BlockSpec((BN, F), lambda i: (i, 0)),
            pl.BlockSpec((F, 2 * H), lambda i: (0, 0)),
        ],
        out_specs=[pl.BlockSpec((BN, H), lambda i: (i, 0))] * 2,
        out_shape=[jax.ShapeDtypeStruct((N, H), F32)] * 2,
    )(x, wcat)


def _agg_combine(p0, p1, d0, d1, b_ref, z_ref):
    deg = d0[0][:, :1] + d1[0][:, :1]
    inv = 1.0 / jnp.maximum(deg, 1.0)
    return (p0[0] + p1[0]) * inv + b_ref[...] + z_ref[...]


def _mid_body(wl, emit_h, p0, p1, d0, d1, z_ref, b_ref, w_ref, *outs):
    h = jnp.maximum(_agg_combine(p0, p1, d0, d1, b_ref, z_ref), 0.0)
    acc = jnp.dot(h, w_ref[...], preferred_element_type=F32)
    outs[0][...] = acc[:, :wl]
    outs[1][...] = acc[:, wl:]
    if emit_h:
        outs[2][...] = h


def _mid(p, d, z, b, wcat, wl, emit_h):
    wtot = wcat.shape[1]
    out_shape = [jax.ShapeDtypeStruct((N, wl), F32),
                 jax.ShapeDtypeStruct((N, wtot - wl), F32)]
    out_specs = [pl.BlockSpec((BN, wl), lambda i: (i, 0)),
                 pl.BlockSpec((BN, wtot - wl), lambda i: (i, 0))]
    if emit_h:
        out_shape.append(jax.ShapeDtypeStruct((N, H), F32))
        out_specs.append(pl.BlockSpec((BN, H), lambda i: (i, 0)))
    return pl.pallas_call(
        functools.partial(_mid_body, wl, emit_h),
        grid=(NB,),
        in_specs=[
            pl.BlockSpec((1, BN, H), lambda i: (0, i, 0)),
            pl.BlockSpec((1, BN, H), lambda i: (1, i, 0)),
            pl.BlockSpec((1, BN, 16), lambda i: (0, i, 0)),
            pl.BlockSpec((1, BN, 16), lambda i: (1, i, 0)),
            pl.BlockSpec((BN, H), lambda i: (i, 0)),
            pl.BlockSpec((1, H), lambda i: (0, 0)),
            pl.BlockSpec((H, wtot), lambda i: (0, 0)),
        ],
        out_specs=out_specs,
        out_shape=out_shape,
    )(p, p, d, d, z, b, wcat)


def _final_body(p0, p1, d0, d1, z_ref, b_ref, out_ref):
    o = _agg_combine(p0, p1, d0, d1, b_ref, z_ref)
    mask = lax.broadcasted_iota(jnp.int32, o.shape, 1) < C
    om = jnp.where(mask, o, -jnp.inf)
    m = jnp.max(om, axis=1, keepdims=True)
    e = jnp.where(mask, jnp.exp(o - m), 0.0)
    lse = jnp.log(jnp.sum(e, axis=1, keepdims=True))
    out_ref[...] = (o - m - lse)[:, :C]


def _final(p, d, z, b):
    return pl.pallas_call(
        _final_body,
        grid=(NB,),
        in_specs=[
            pl.BlockSpec((1, BN, CP), lambda i: (0, i, 0)),
            pl.BlockSpec((1, BN, CP), lambda i: (1, i, 0)),
            pl.BlockSpec((1, BN, 16), lambda i: (0, i, 0)),
            pl.BlockSpec((1, BN, 16), lambda i: (1, i, 0)),
            pl.BlockSpec((BN, CP), lambda i: (i, 0)),
            pl.BlockSpec((1, CP), lambda i: (0, 0)),
        ],
        out_specs=pl.BlockSpec((BN, C), lambda i: (i, 0)),
        out_shape=jax.ShapeDtypeStruct((N, C), F32),
    )(p, p, d, d, z, b)


def kernel(x, edge_index, W1l, b1, W1r, W2l, b2, W2r, W3l, b3, W3r):
    src = edge_index[0]
    dst = edge_index[1]
    pad = NW * EWP - E
    src3 = jnp.pad(src, (0, pad)).reshape(NW, CH, K).astype(jnp.int32)
    dst3 = jnp.pad(dst, (0, pad), constant_values=N).reshape(NW, CH, K).astype(jnp.int32)

    zeros_h = jnp.zeros((RPT, H), F32)
    zeros_c = jnp.zeros((RPT, CP), F32)
    zeros16 = jnp.zeros((RPT, 16), F32)
    ones16 = jnp.ones((K, 16), F32)

    w1 = jnp.concatenate([W1l, W1r], axis=0).T
    w2 = jnp.concatenate([W2l, W2r], axis=0).T
    w3l = jnp.pad(W3l, ((0, CP - C), (0, 0)))
    w3r = jnp.pad(W3r, ((0, CP - C), (0, 0)))
    w3 = jnp.concatenate([w3l, w3r], axis=0).T
    b1r = b1.reshape(1, H)
    b2r = b2.reshape(1, H)
    b3r = jnp.pad(b3, (0, CP - C)).reshape(1, CP)

    deg = _deg_kernel(dst3, zeros16, ones16)
    y1, z1 = _entry(x, w1)
    p1 = _segsum_h(y1, src3, dst3, zeros_h)
    y2, z2 = _mid(p1, deg, z1, b1r, w2, H, False)
    p2 = _segsum_h(y2, src3, dst3, zeros_h)
    y3, z3, h2 = _mid(p2, deg, z2, b2r, w3, CP, True)
    p3 = _segsum_c(y3, src3, dst3, zeros_c)
    logp = _final(p3, deg, z3, b3r)
    return logp, h2
